# baseline (speedup 1.0000x reference)
import functools

import jax
import jax.numpy as jnp
from jax import lax
from jax.experimental import pallas as pl
from jax.experimental.pallas import tpu as pltpu
from jax.experimental.pallas import tpu_sc as plsc

N = 255
NNZ_PAD = 768
TBL = 2048


def _sc_extract(coo_packed):
    mesh = plsc.VectorSubcoreMesh(core_axis_name="c", subcore_axis_name="s")

    @functools.partial(
        pl.kernel,
        out_type=jax.ShapeDtypeStruct((2 * TBL,), jnp.float32),
        mesh=mesh,
        compiler_params=pltpu.CompilerParams(needs_layout_passes=False),
        scratch_types=[
            pltpu.VMEM((6, NNZ_PAD), jnp.int32),
            pltpu.VMEM((2 * TBL,), jnp.float32),
        ],
    )
    def k(coo_hbm, out_hbm, coo_v, tbl_v):
        @pl.when((lax.axis_index("c") == 0) & (lax.axis_index("s") == 0))
        def _():
            pltpu.sync_copy(coo_hbm, coo_v)
            zero = jnp.zeros((16,), jnp.float32)
            for j in range(2 * TBL // 16):
                tbl_v[pl.ds(j * 16, 16)] = zero
            for m in range(2):
                base = m * TBL
                for j in range(NNZ_PAD // 16):
                    r = coo_v[3 * m, pl.ds(j * 16, 16)]
                    c = coo_v[3 * m + 1, pl.ds(j * 16, 16)]
                    v = plsc.bitcast(coo_v[3 * m + 2, pl.ds(j * 16, 16)],
                                     jnp.float32)
                    idx = base + r * 8 + (c - r + 1)
                    plsc.store_scatter(tbl_v, [idx], v)
            pltpu.sync_copy(tbl_v, out_hbm)

    return k(coo_packed)

    return k(rows_sm, cols_sm, vals_sm, rows_sp, cols_sp, vals_sp)


def _band_apply(coef, y):
    return (coef[:, 0:1] * jnp.roll(y, 1, axis=0)
            + coef[:, 1:2] * y
            + coef[:, 2:3] * jnp.roll(y, -1, axis=0))


def _make_body(bc):
    def tile_period(tbl_ref):
        return jnp.concatenate([tbl_ref[...][:N]] * bc, axis=0)

    def body(tbl_sm, tbl_sp, h_ref,
             We0, be0, We1, be1, We2, be2, Wd0, bd0, Wd1, bd1, Wd2, bd2,
             out_ref, coef_sm, coef_sp):
        @pl.when(pl.program_id(0) == 0)
        def _():
            coef_sm[...] = tile_period(tbl_sm)
            coef_sp[...] = tile_period(tbl_sp)

        csm = coef_sm[...]
        csp = coef_sp[...]
        x = h_ref[...].reshape(bc * N, 2)
        for W, b, coef in ((We0, be0, csm), (We1, be1, csm), (We2, be2, csm),
                           (Wd0, bd0, csp), (Wd1, bd1, csp), (Wd2, bd2, csp)):
            y = jnp.dot(x, W[...], preferred_element_type=jnp.float32) + b[...]
            x = jnp.maximum(_band_apply(coef, y), 0.0)
        out_ref[...] = x.reshape(bc, N, 2)

    return body


def kernel(H, We0, be0, We1, be1, We2, be2, Wd0, bd0, Wd1, bd1, Wd2, bd2,
           sm_rows, sm_cols, sm_vals, sp_rows, sp_cols, sp_vals):
    B = H.shape[0]
    bc = 16
    blk = bc * N

    def pad_nnz(a):
        pad = NNZ_PAD - a.shape[0]
        if a.dtype == jnp.int32:
            return jnp.pad(a, (0, pad), constant_values=N)
        return jnp.pad(a.view(jnp.int32), (0, pad))

    coo_packed = jnp.stack([pad_nnz(a) for a in
                            (sm_rows, sm_cols, sm_vals,
                             sp_rows, sp_cols, sp_vals)])
    tbl = _sc_extract(coo_packed)
    tbl_sm = tbl[:TBL].reshape(TBL // 8, 8)
    tbl_sp = tbl[TBL:].reshape(TBL // 8, 8)

    weights = [We0, We1, We2, Wd0, Wd1, Wd2]
    biases = [b.reshape(1, -1) for b in (be0, be1, be2, bd0, bd1, bd2)]

    full = lambda a: pl.BlockSpec(a.shape, lambda i: (0, 0))
    in_specs = [full(tbl_sm), full(tbl_sp),
                pl.BlockSpec((bc, N, 2), lambda i: (i, 0, 0))]
    for W, b in zip(weights, biases):
        in_specs += [full(W), full(b)]

    inputs = [tbl_sm, tbl_sp, H]
    for W, b in zip(weights, biases):
        inputs += [W, b]

    return pl.pallas_call(
        _make_body(bc),
        grid=(B // bc,),
        in_specs=in_specs,
        out_specs=pl.BlockSpec((bc, N, 2), lambda i: (i, 0, 0)),
        out_shape=jax.ShapeDtypeStruct((B, N, 2), jnp.float32),
        scratch_shapes=[pltpu.VMEM((blk, 8), jnp.float32),
                        pltpu.VMEM((blk, 8), jnp.float32)],
    )(*inputs)


# speedup vs baseline: 1.0845x; 1.0845x over previous
import functools

import jax
import jax.numpy as jnp
from jax import lax
from jax.experimental import pallas as pl
from jax.experimental.pallas import tpu as pltpu
from jax.experimental.pallas import tpu_sc as plsc

N = 255
NNZ_PAD = 768
TBL = 2048


def _sc_extract(coo_packed):
    mesh = plsc.VectorSubcoreMesh(core_axis_name="c", subcore_axis_name="s")

    @functools.partial(
        pl.kernel,
        out_type=jax.ShapeDtypeStruct((2 * TBL,), jnp.float32),
        mesh=mesh,
        compiler_params=pltpu.CompilerParams(needs_layout_passes=False),
        scratch_types=[
            pltpu.VMEM((6, NNZ_PAD), jnp.int32),
            pltpu.VMEM((2 * TBL,), jnp.float32),
        ],
    )
    def k(coo_hbm, out_hbm, coo_v, tbl_v):
        @pl.when((lax.axis_index("c") == 0) & (lax.axis_index("s") == 0))
        def _():
            pltpu.sync_copy(coo_hbm, coo_v)
            zero = jnp.zeros((16,), jnp.float32)
            for j in range(2 * TBL // 16):
                tbl_v[pl.ds(j * 16, 16)] = zero
            for m in range(2):
                base = m * TBL
                for j in range(NNZ_PAD // 16):
                    r = coo_v[3 * m, pl.ds(j * 16, 16)]
                    c = coo_v[3 * m + 1, pl.ds(j * 16, 16)]
                    v = plsc.bitcast(coo_v[3 * m + 2, pl.ds(j * 16, 16)],
                                     jnp.float32)
                    idx = base + r * 8 + (c - r + 1)
                    plsc.store_scatter(tbl_v, [idx], v)
            pltpu.sync_copy(tbl_v, out_hbm)

    return k(coo_packed)

    return k(rows_sm, cols_sm, vals_sm, rows_sp, cols_sp, vals_sp)


def _band_apply(coef, y):
    return (coef[:, 0:1] * jnp.roll(y, 1, axis=0)
            + coef[:, 1:2] * y
            + coef[:, 2:3] * jnp.roll(y, -1, axis=0))


def _make_body(bc):
    def tile_period(tbl_ref):
        return jnp.concatenate([tbl_ref[...][:N]] * bc, axis=0)

    def body(tbl_sm, tbl_sp, h_ref,
             We0, be0, We1, be1, We2, be2, Wd0, bd0, Wd1, bd1, Wd2, bd2,
             out_ref, coef_sm, coef_sp):
        @pl.when(pl.program_id(0) == 0)
        def _():
            coef_sm[...] = tile_period(tbl_sm)
            coef_sp[...] = tile_period(tbl_sp)

        csm = coef_sm[...]
        csp = coef_sp[...]
        x = h_ref[...]
        for W, b, coef in ((We0, be0, csm), (We1, be1, csm), (We2, be2, csm),
                           (Wd0, bd0, csp), (Wd1, bd1, csp), (Wd2, bd2, csp)):
            y = jnp.dot(x, W[...], preferred_element_type=jnp.float32) + b[...]
            x = jnp.maximum(_band_apply(coef, y), 0.0)
        out_ref[...] = x

    return body


def kernel(H, We0, be0, We1, be1, We2, be2, Wd0, bd0, Wd1, bd1, Wd2, bd2,
           sm_rows, sm_cols, sm_vals, sp_rows, sp_cols, sp_vals):
    B = H.shape[0]
    bc = 16
    blk = bc * N

    def pad_nnz(a):
        pad = NNZ_PAD - a.shape[0]
        if a.dtype == jnp.int32:
            return jnp.pad(a, (0, pad), constant_values=N)
        return jnp.pad(a.view(jnp.int32), (0, pad))

    coo_packed = jnp.stack([pad_nnz(a) for a in
                            (sm_rows, sm_cols, sm_vals,
                             sp_rows, sp_cols, sp_vals)])
    tbl = _sc_extract(coo_packed)
    tbl_sm = tbl[:TBL].reshape(TBL // 8, 8)
    tbl_sp = tbl[TBL:].reshape(TBL // 8, 8)

    weights = [We0, We1, We2, Wd0, Wd1, Wd2]
    biases = [b.reshape(1, -1) for b in (be0, be1, be2, bd0, bd1, bd2)]

    full = lambda a: pl.BlockSpec(a.shape, lambda i: (0, 0))
    in_specs = [full(tbl_sm), full(tbl_sp),
                pl.BlockSpec((blk, 2), lambda i: (i, 0))]
    for W, b in zip(weights, biases):
        in_specs += [full(W), full(b)]

    inputs = [tbl_sm, tbl_sp, H.reshape(B * N, 2)]
    for W, b in zip(weights, biases):
        inputs += [W, b]

    out = pl.pallas_call(
        _make_body(bc),
        grid=(B // bc,),
        in_specs=in_specs,
        out_specs=pl.BlockSpec((blk, 2), lambda i: (i, 0)),
        out_shape=jax.ShapeDtypeStruct((B * N, 2), jnp.float32),
        scratch_shapes=[pltpu.VMEM((blk, 8), jnp.float32),
                        pltpu.VMEM((blk, 8), jnp.float32)],
    )(*inputs)
    return out.reshape(B, N, 2)
